# baseline (speedup 1.0000x reference)
import jax
import jax.numpy as jnp
from jax.experimental import pallas as pl
from jax.experimental.pallas import tpu as pltpu

_EPS = 1e-9
_BLOCK = 4096


def _conv_body(p1_ref, p2_ref, out_ref):
    a = p1_ref[...]
    b = p2_ref[...]
    r = a.shape[0]
    bp = jnp.concatenate([b, jnp.zeros((r, 9), b.dtype)], axis=1)
    acc = None
    for i in range(10):
        ai = jnp.broadcast_to(a[:, i : i + 1], (r, 19))
        prod = ai * bp
        if i:
            prod = jnp.concatenate([prod[:, 19 - i :], prod[:, : 19 - i]], axis=1)
        acc = prod if acc is None else acc + prod
    tot = jnp.sum(acc, axis=1, keepdims=True)
    out_ref[...] = acc / (tot + _EPS)


def kernel(p1, p2):
    B = p1.shape[0]
    nblk = pl.cdiv(B, _BLOCK)
    Bp = nblk * _BLOCK
    if Bp != B:
        p1 = jnp.pad(p1, ((0, Bp - B), (0, 0)))
        p2 = jnp.pad(p2, ((0, Bp - B), (0, 0)))
    out = pl.pallas_call(
        _conv_body,
        out_shape=jax.ShapeDtypeStruct((Bp, 19), p1.dtype),
        grid=(nblk,),
        in_specs=[
            pl.BlockSpec((_BLOCK, 10), lambda i: (i, 0)),
            pl.BlockSpec((_BLOCK, 10), lambda i: (i, 0)),
        ],
        out_specs=pl.BlockSpec((_BLOCK, 19), lambda i: (i, 0)),
        compiler_params=pltpu.CompilerParams(
            dimension_semantics=("parallel",),
        ),
        name="digit_sum_conv",
    )(p1, p2)
    return out[:B] if Bp != B else out
